# Initial kernel scaffold; baseline (speedup 1.0000x reference)
import jax
import jax.numpy as jnp
from jax.experimental import pallas as pl


def kernel(x, W1, b1, W2, b2):
    raise NotImplementedError("write your pallas kernel here")


# speedup vs baseline: 2.7745x; 2.7745x over previous
import functools

import jax
import jax.numpy as jnp
from jax.experimental import pallas as pl

_FEATURE_DIM = 4096
_HIDDEN_DIM = 256
_N_EXPERTS = 64
_TOKENS = 16384
_BT = 512


def _gate_body(x_ref, w1_ref, b1_ref, w2_ref, b2_ref, gates_ref, idx_ref):
    h = jnp.dot(x_ref[...], w1_ref[...], preferred_element_type=jnp.float32)
    h = jnp.maximum(h + b1_ref[...], 0.0)
    logits = jnp.dot(h, w2_ref[...], preferred_element_type=jnp.float32)
    logits = logits + b2_ref[...]

    iota = jax.lax.broadcasted_iota(jnp.int32, logits.shape, 1)
    m1 = jnp.max(logits, axis=1, keepdims=True)
    i1 = jnp.min(jnp.where(logits == m1, iota, _N_EXPERTS), axis=1, keepdims=True)
    first = iota == i1
    masked = jnp.where(first, -jnp.inf, logits)
    m2 = jnp.max(masked, axis=1, keepdims=True)
    i2 = jnp.min(jnp.where(masked == m2, iota, _N_EXPERTS), axis=1, keepdims=True)

    g1 = jax.nn.sigmoid(m1 - m2)
    g2 = 1.0 - g1
    gates_ref[...] = jnp.where(first, g1, jnp.where(iota == i2, g2, 0.0))

    iota2 = jax.lax.broadcasted_iota(jnp.int32, idx_ref.shape, 1)
    idx_ref[...] = jnp.where(iota2 == 0, i1, i2)


@jax.jit
def kernel(x, W1, b1, W2, b2):
    grid = (_TOKENS // _BT,)
    gates, idx = pl.pallas_call(
        _gate_body,
        grid=grid,
        in_specs=[
            pl.BlockSpec((_BT, _FEATURE_DIM), lambda i: (i, 0)),
            pl.BlockSpec((_FEATURE_DIM, _HIDDEN_DIM), lambda i: (0, 0)),
            pl.BlockSpec((1, _HIDDEN_DIM), lambda i: (0, 0)),
            pl.BlockSpec((_HIDDEN_DIM, _N_EXPERTS), lambda i: (0, 0)),
            pl.BlockSpec((1, _N_EXPERTS), lambda i: (0, 0)),
        ],
        out_specs=[
            pl.BlockSpec((_BT, _N_EXPERTS), lambda i: (i, 0)),
            pl.BlockSpec((_BT, 2), lambda i: (i, 0)),
        ],
        out_shape=[
            jax.ShapeDtypeStruct((_TOKENS, _N_EXPERTS), jnp.float32),
            jax.ShapeDtypeStruct((_TOKENS, 2), jnp.int32),
        ],
    )(x, W1, b1.reshape(1, -1), W2, b2.reshape(1, -1))
    return (gates, idx)


# speedup vs baseline: 3.0002x; 1.0813x over previous
import functools

import jax
import jax.numpy as jnp
from jax.experimental import pallas as pl

_FEATURE_DIM = 4096
_HIDDEN_DIM = 256
_N_EXPERTS = 64
_TOKENS = 16384
_BT = 1024


def _gate_body(x_ref, w1_ref, b1_ref, w2_ref, b2_ref, gates_ref, idx_ref):
    h = jnp.dot(x_ref[...], w1_ref[...], preferred_element_type=jnp.float32)
    h = jnp.maximum(h + b1_ref[...], 0.0)
    logits = jnp.dot(h, w2_ref[...], preferred_element_type=jnp.float32)
    logits = logits + b2_ref[...]

    iota = jax.lax.broadcasted_iota(jnp.int32, logits.shape, 1)
    m1 = jnp.max(logits, axis=1, keepdims=True)
    i1 = jnp.min(jnp.where(logits == m1, iota, _N_EXPERTS), axis=1, keepdims=True)
    first = iota == i1
    masked = jnp.where(first, -jnp.inf, logits)
    m2 = jnp.max(masked, axis=1, keepdims=True)
    i2 = jnp.min(jnp.where(masked == m2, iota, _N_EXPERTS), axis=1, keepdims=True)

    g1 = jax.nn.sigmoid(m1 - m2)
    g2 = 1.0 - g1
    gates_ref[...] = jnp.where(first, g1, jnp.where(iota == i2, g2, 0.0))

    iota2 = jax.lax.broadcasted_iota(jnp.int32, idx_ref.shape, 1)
    idx_ref[...] = jnp.where(iota2 == 0, i1, i2)


@jax.jit
def kernel(x, W1, b1, W2, b2):
    grid = (_TOKENS // _BT,)
    gates, idx = pl.pallas_call(
        _gate_body,
        grid=grid,
        in_specs=[
            pl.BlockSpec((_BT, _FEATURE_DIM), lambda i: (i, 0)),
            pl.BlockSpec((_FEATURE_DIM, _HIDDEN_DIM), lambda i: (0, 0)),
            pl.BlockSpec((1, _HIDDEN_DIM), lambda i: (0, 0)),
            pl.BlockSpec((_HIDDEN_DIM, _N_EXPERTS), lambda i: (0, 0)),
            pl.BlockSpec((1, _N_EXPERTS), lambda i: (0, 0)),
        ],
        out_specs=[
            pl.BlockSpec((_BT, _N_EXPERTS), lambda i: (i, 0)),
            pl.BlockSpec((_BT, 2), lambda i: (i, 0)),
        ],
        out_shape=[
            jax.ShapeDtypeStruct((_TOKENS, _N_EXPERTS), jnp.float32),
            jax.ShapeDtypeStruct((_TOKENS, 2), jnp.int32),
        ],
    )(x, W1, b1.reshape(1, -1), W2, b2.reshape(1, -1))
    return (gates, idx)


# speedup vs baseline: 3.0217x; 1.0072x over previous
import jax
import jax.numpy as jnp
from jax.experimental import pallas as pl

_FEATURE_DIM = 4096
_HALF = _FEATURE_DIM // 2
_HIDDEN_DIM = 256
_N_EXPERTS = 64
_TOKENS = 16384
_BT = 1024


def _gate_body(xa_ref, xb_ref, w1a_ref, w1b_ref, b1_ref, w2_ref, b2_ref,
               gates_ref, idx_ref):
    h = jnp.dot(xa_ref[...], w1a_ref[...], preferred_element_type=jnp.float32)
    h = h + jnp.dot(xb_ref[...], w1b_ref[...], preferred_element_type=jnp.float32)
    h = jnp.maximum(h + b1_ref[...], 0.0)
    logits = jnp.dot(h, w2_ref[...], preferred_element_type=jnp.float32)
    logits = logits + b2_ref[...]

    iota = jax.lax.broadcasted_iota(jnp.int32, logits.shape, 1)
    m1 = jnp.max(logits, axis=1, keepdims=True)
    i1 = jnp.min(jnp.where(logits == m1, iota, _N_EXPERTS), axis=1, keepdims=True)
    first = iota == i1
    masked = jnp.where(first, -jnp.inf, logits)
    m2 = jnp.max(masked, axis=1, keepdims=True)
    i2 = jnp.min(jnp.where(masked == m2, iota, _N_EXPERTS), axis=1, keepdims=True)

    g1 = jax.nn.sigmoid(m1 - m2)
    g2 = 1.0 - g1
    gates_ref[...] = jnp.where(first, g1, jnp.where(iota == i2, g2, 0.0))

    iota2 = jax.lax.broadcasted_iota(jnp.int32, idx_ref.shape, 1)
    idx_ref[...] = jnp.where(iota2 == 0, i1, i2)


@jax.jit
def kernel(x, W1, b1, W2, b2):
    grid = (_TOKENS // _BT,)
    gates, idx = pl.pallas_call(
        _gate_body,
        grid=grid,
        in_specs=[
            pl.BlockSpec((_BT, _HALF), lambda i: (i, 0)),
            pl.BlockSpec((_BT, _HALF), lambda i: (i, 1)),
            pl.BlockSpec((_HALF, _HIDDEN_DIM), lambda i: (0, 0)),
            pl.BlockSpec((_HALF, _HIDDEN_DIM), lambda i: (1, 0)),
            pl.BlockSpec((1, _HIDDEN_DIM), lambda i: (0, 0)),
            pl.BlockSpec((_HIDDEN_DIM, _N_EXPERTS), lambda i: (0, 0)),
            pl.BlockSpec((1, _N_EXPERTS), lambda i: (0, 0)),
        ],
        out_specs=[
            pl.BlockSpec((_BT, _N_EXPERTS), lambda i: (i, 0)),
            pl.BlockSpec((_BT, 2), lambda i: (i, 0)),
        ],
        out_shape=[
            jax.ShapeDtypeStruct((_TOKENS, _N_EXPERTS), jnp.float32),
            jax.ShapeDtypeStruct((_TOKENS, 2), jnp.int32),
        ],
    )(x, x, W1, W1, b1.reshape(1, -1), W2, b2.reshape(1, -1))
    return (gates, idx)
